# baseline (speedup 1.0000x reference)
import jax
import jax.numpy as jnp
from jax.experimental import pallas as pl
from jax.experimental.pallas import tpu as pltpu

SEQ_BLOCK = 64


def _body(pos_ref, x_ref, pe_ref, tab_ref, o_ref):
    i = pl.program_id(0)
    s_blk = x_ref.shape[0]
    batch = x_ref.shape[1]
    s_ids = i * s_blk + jax.lax.broadcasted_iota(jnp.int32, (s_blk, 1), 0)
    c = jnp.zeros((s_blk, batch), jnp.float32)
    for k in range(pos_ref.shape[0]):
        pk = pos_ref[k, :].reshape(1, batch)
        c = c + (s_ids == pk).astype(jnp.float32)
    pe = pe_ref[...]
    tab = tab_ref[...]
    o_ref[...] = x_ref[...] + pe[:, None, :] + c[:, :, None] * tab[:, None, :]


def kernel(x, peak_positions, pe, peak_table):
    seq_len, batch, dim = x.shape
    num_peaks = peak_positions.shape[1]
    pos_t = peak_positions.T
    grid = (seq_len // SEQ_BLOCK,)
    return pl.pallas_call(
        _body,
        grid=grid,
        in_specs=[
            pl.BlockSpec((num_peaks, batch), lambda i: (0, 0)),
            pl.BlockSpec((SEQ_BLOCK, batch, dim), lambda i: (i, 0, 0)),
            pl.BlockSpec((SEQ_BLOCK, dim), lambda i: (i, 0)),
            pl.BlockSpec((SEQ_BLOCK, dim), lambda i: (i, 0)),
        ],
        out_specs=pl.BlockSpec((SEQ_BLOCK, batch, dim), lambda i: (i, 0, 0)),
        out_shape=jax.ShapeDtypeStruct(x.shape, x.dtype),
        compiler_params=pltpu.CompilerParams(
            dimension_semantics=("parallel",),
        ),
    )(pos_t, x, pe[:seq_len], peak_table[:seq_len])
